# baseline (speedup 1.0000x reference)
import functools

import jax
import jax.numpy as jnp
from jax import lax
from jax.experimental import pallas as pl
from jax.experimental.pallas import tpu as pltpu
from jax.experimental.pallas import tpu_sc as plsc

N_NODES = 10000
D = 128
N_EDGES = 320000

NC = 2
NS = 16
LANES = 16

CHUNK = 128
CPP = 160
N_CHUNKS = CPP * NS
E_PAD = N_CHUNKS * CHUNK
CF = 44
CS = CPP - CF
ROWS_PER_TILE = 624
EXTRA_BASE = ROWS_PER_TILE * NS


def _sc_body(src_h, dst_h, val_h, x_h, out_h,
             acc, srcv, dstv, valv, rows, gsem):
    c = lax.axis_index("c")
    s = lax.axis_index("s")
    rb = s * ROWS_PER_TILE

    @pl.loop(0, CHUNK)
    def _(i):
        for cv in range(D // LANES):
            rows[i, pl.ds(cv * LANES, LANES)] = jnp.zeros((LANES,), jnp.float32)

    for k in range(ROWS_PER_TILE // CHUNK):
        pltpu.sync_copy(rows, acc.at[pl.ds(rb + k * CHUNK, CHUNK)])
    rem = ROWS_PER_TILE % CHUNK
    pltpu.sync_copy(rows.at[pl.ds(0, rem)],
                    acc.at[pl.ds(rb + ROWS_PER_TILE - rem, rem)])

    @pl.when(s == NS - 1)
    def _():
        pltpu.sync_copy(rows.at[pl.ds(0, N_NODES - EXTRA_BASE)],
                        acc.at[pl.ds(EXTRA_BASE, N_NODES - EXTRA_BASE)])

    plsc.subcore_barrier()

    cbase = s * CPP + jnp.where(c == 0, 0, CF)
    n_my = jnp.where(c == 0, CF, CS)

    @pl.loop(0, n_my)
    def _(j):
        ebase = (cbase + j) * CHUNK
        pltpu.sync_copy(src_h.at[pl.ds(ebase, CHUNK)], srcv)
        pltpu.sync_copy(dst_h.at[pl.ds(ebase, CHUNK)], dstv.at[0])
        pltpu.sync_copy(val_h.at[pl.ds(ebase, CHUNK)], valv)
        pltpu.async_copy(x_h.at[srcv], rows, gsem).wait()

        @pl.loop(0, CHUNK // LANES)
        def _(g):
            vv = valv[pl.ds(g * LANES, LANES)]
            for l in range(LANES):
                v = vv[l]
                e = g * LANES + l
                for cv in range(D // LANES):
                    sl = pl.ds(cv * LANES, LANES)
                    rows[e, sl] = rows[e, sl] * v

        pltpu.sync_copy(rows, acc.at[dstv.at[0]], add=True)

    plsc.subcore_barrier()
    pltpu.sync_copy(acc.at[pl.ds(rb, ROWS_PER_TILE)],
                    out_h.at[c, pl.ds(rb, ROWS_PER_TILE)])

    @pl.when(s == NS - 1)
    def _():
        pltpu.sync_copy(acc.at[pl.ds(EXTRA_BASE, N_NODES - EXTRA_BASE)],
                        out_h.at[c, pl.ds(EXTRA_BASE, N_NODES - EXTRA_BASE)])


_sc_scatter = pl.kernel(
    _sc_body,
    out_type=jax.ShapeDtypeStruct((NC, N_NODES, D), jnp.float32),
    mesh=plsc.VectorSubcoreMesh(
        core_axis_name="c", subcore_axis_name="s",
        num_cores=NC, num_subcores=NS),
    scratch_types=[
        pltpu.VMEM_SHARED((N_NODES, D), jnp.float32),
        pltpu.VMEM((CHUNK,), jnp.int32),
        pltpu.VMEM((1, CHUNK), jnp.int32),
        pltpu.VMEM((CHUNK,), jnp.float32),
        pltpu.VMEM((CHUNK, D), jnp.float32),
        pltpu.SemaphoreType.DMA,
    ],
)


BR = 400


def _mm_body(p_ref, w_ref, b_ref, o_ref):
    z = p_ref[0] + p_ref[1]
    o_ref[...] = (
        jnp.dot(z, w_ref[...], preferred_element_type=jnp.float32) + b_ref[...]
    )


_tc_matmul = pl.pallas_call(
    _mm_body,
    grid=(N_NODES // BR,),
    in_specs=[
        pl.BlockSpec((NC, BR, D), lambda i: (0, i, 0)),
        pl.BlockSpec((D, D), lambda i: (0, 0)),
        pl.BlockSpec((1, D), lambda i: (0, 0)),
    ],
    out_specs=pl.BlockSpec((BR, D), lambda i: (i, 0)),
    out_shape=jax.ShapeDtypeStruct((N_NODES, D), jnp.float32),
)


@jax.jit
def kernel(adjacency_indices, adjacency_values, input_features, W, bias):
    dst = adjacency_indices[0]
    src = adjacency_indices[1]
    pad = E_PAD - N_EDGES
    src_p = jnp.concatenate([src, jnp.zeros((pad,), jnp.int32)])
    dst_p = jnp.concatenate([dst, jnp.zeros((pad,), jnp.int32)])
    val_p = jnp.concatenate([adjacency_values, jnp.zeros((pad,), jnp.float32)])
    partials = _sc_scatter(src_p, dst_p, val_p, input_features)
    return _tc_matmul(partials, W, bias.reshape(1, D))


# speedup vs baseline: 1.3932x; 1.3932x over previous
import functools

import jax
import jax.numpy as jnp
from jax import lax
from jax.experimental import pallas as pl
from jax.experimental.pallas import tpu as pltpu
from jax.experimental.pallas import tpu_sc as plsc

N_NODES = 10000
D = 128
N_EDGES = 320000

NC = 2
NS = 16
LANES = 16

CHUNK = 128
CPW = 80
N_CHUNKS = CPW * NC * NS
E_PAD = N_CHUNKS * CHUNK
ROWS_PER_TILE = 624
EXTRA_BASE = ROWS_PER_TILE * NS


def _sc_body(src_h, dst_h, val_h, x_h, out_h,
             acc, srcv, dstv, valv, rows, isems, gsem):
    c = lax.axis_index("c")
    s = lax.axis_index("s")
    rb = s * ROWS_PER_TILE

    @pl.loop(0, CHUNK)
    def _(i):
        for cv in range(D // LANES):
            rows[i, pl.ds(cv * LANES, LANES)] = jnp.zeros((LANES,), jnp.float32)

    for k in range(ROWS_PER_TILE // CHUNK):
        pltpu.sync_copy(rows, acc.at[pl.ds(rb + k * CHUNK, CHUNK)])
    rem = ROWS_PER_TILE % CHUNK
    pltpu.sync_copy(rows.at[pl.ds(0, rem)],
                    acc.at[pl.ds(rb + ROWS_PER_TILE - rem, rem)])

    @pl.when(s == NS - 1)
    def _():
        pltpu.sync_copy(rows.at[pl.ds(0, N_NODES - EXTRA_BASE)],
                        acc.at[pl.ds(EXTRA_BASE, N_NODES - EXTRA_BASE)])

    plsc.subcore_barrier()

    wid = c * NS + s
    cbase = wid * CPW

    def fire_idx(j, t):
        ebase = (cbase + j) * CHUNK
        sem = isems.at[t]
        pltpu.async_copy(src_h.at[pl.ds(ebase, CHUNK)], srcv.at[t], sem)
        pltpu.async_copy(dst_h.at[pl.ds(ebase, CHUNK)], dstv.at[t], sem)
        pltpu.async_copy(val_h.at[pl.ds(ebase, CHUNK)], valv.at[t], sem)

    def drain_idx(j, t):
        ebase = (cbase + j) * CHUNK
        sem = isems.at[t]
        pltpu.make_async_copy(src_h.at[pl.ds(ebase, CHUNK)], srcv.at[t], sem).wait()
        pltpu.make_async_copy(dst_h.at[pl.ds(ebase, CHUNK)], dstv.at[t], sem).wait()
        pltpu.make_async_copy(val_h.at[pl.ds(ebase, CHUNK)], valv.at[t], sem).wait()

    def chunk_step(j, t):
        drain_idx(j, t)
        pltpu.async_copy(x_h.at[srcv.at[t]], rows, gsem)

        @pl.when(j + 1 < CPW)
        def _():
            fire_idx(j + 1, 1 - t)

        pltpu.make_async_copy(x_h.at[srcv.at[t]], rows, gsem).wait()

        @pl.loop(0, CHUNK // LANES)
        def _(g):
            vv = valv[t, pl.ds(g * LANES, LANES)]
            for l in range(LANES):
                v = vv[l]
                e = g * LANES + l
                for cv in range(D // LANES):
                    sl = pl.ds(cv * LANES, LANES)
                    rows[e, sl] = rows[e, sl] * v

        pltpu.sync_copy(rows, acc.at[dstv.at[t]], add=True)

    fire_idx(0, 0)

    @pl.loop(0, CPW // 2)
    def _(j2):
        chunk_step(2 * j2, 0)
        chunk_step(2 * j2 + 1, 1)

    plsc.subcore_barrier()
    pltpu.sync_copy(acc.at[pl.ds(rb, ROWS_PER_TILE)],
                    out_h.at[c, pl.ds(rb, ROWS_PER_TILE)])

    @pl.when(s == NS - 1)
    def _():
        pltpu.sync_copy(acc.at[pl.ds(EXTRA_BASE, N_NODES - EXTRA_BASE)],
                        out_h.at[c, pl.ds(EXTRA_BASE, N_NODES - EXTRA_BASE)])


_sc_scatter = pl.kernel(
    _sc_body,
    out_type=jax.ShapeDtypeStruct((NC, N_NODES, D), jnp.float32),
    mesh=plsc.VectorSubcoreMesh(
        core_axis_name="c", subcore_axis_name="s",
        num_cores=NC, num_subcores=NS),
    scratch_types=[
        pltpu.VMEM_SHARED((N_NODES, D), jnp.float32),
        pltpu.VMEM((2, CHUNK), jnp.int32),
        pltpu.VMEM((2, CHUNK), jnp.int32),
        pltpu.VMEM((2, CHUNK), jnp.float32),
        pltpu.VMEM((CHUNK, D), jnp.float32),
        pltpu.SemaphoreType.DMA((2,)),
        pltpu.SemaphoreType.DMA,
    ],
)


BR = 400


def _mm_body(p_ref, w_ref, b_ref, o_ref):
    z = p_ref[0] + p_ref[1]
    o_ref[...] = (
        jnp.dot(z, w_ref[...], preferred_element_type=jnp.float32) + b_ref[...]
    )


_tc_matmul = pl.pallas_call(
    _mm_body,
    grid=(N_NODES // BR,),
    in_specs=[
        pl.BlockSpec((NC, BR, D), lambda i: (0, i, 0)),
        pl.BlockSpec((D, D), lambda i: (0, 0)),
        pl.BlockSpec((1, D), lambda i: (0, 0)),
    ],
    out_specs=pl.BlockSpec((BR, D), lambda i: (i, 0)),
    out_shape=jax.ShapeDtypeStruct((N_NODES, D), jnp.float32),
)


@jax.jit
def kernel(adjacency_indices, adjacency_values, input_features, W, bias):
    dst = adjacency_indices[0]
    src = adjacency_indices[1]
    pad = E_PAD - N_EDGES
    src_p = jnp.concatenate([src, jnp.zeros((pad,), jnp.int32)])
    dst_p = jnp.concatenate([dst, jnp.zeros((pad,), jnp.int32)])
    val_p = jnp.concatenate([adjacency_values, jnp.zeros((pad,), jnp.float32)])
    partials = _sc_scatter(src_p, dst_p, val_p, input_features)
    return _tc_matmul(partials, W, bias.reshape(1, D))


# speedup vs baseline: 2.0010x; 1.4362x over previous
import functools

import jax
import jax.numpy as jnp
from jax import lax
from jax.experimental import pallas as pl
from jax.experimental.pallas import tpu as pltpu
from jax.experimental.pallas import tpu_sc as plsc

N_NODES = 10000
D = 128
N_EDGES = 320000

NC = 2
NS = 16
NW = NC * NS
LANES = 16

EDGES_PER_WORKER = N_EDGES // NW
CHUNK = 128
FULL_CHUNKS = EDGES_PER_WORKER // CHUNK
TAIL = EDGES_PER_WORKER - FULL_CHUNKS * CHUNK
ROWS_PER_TILE = 624
ZCHUNK = 208
EXTRA_BASE = ROWS_PER_TILE * NS


def _scale_rows(rows_ref, val_ref, n_edges):
    @pl.loop(0, n_edges // LANES)
    def _(g):
        vv = val_ref[pl.ds(g * LANES, LANES)]
        for l in range(LANES):
            v = vv[l]
            e = g * LANES + l
            for c in range(D // LANES):
                sl = pl.ds(c * LANES, LANES)
                rows_ref[e, sl] = rows_ref[e, sl] * v


def _sc_body(src_h, dst_h, val_h, x_h, out_h,
             acc, zbuf, srcv, dstv, valv, rows,
             srct, dstt, valt, rowst, gsem):
    c = lax.axis_index("c")
    s = lax.axis_index("s")
    wid = c * NS + s
    ebase = wid * EDGES_PER_WORKER

    @pl.loop(0, ZCHUNK)
    def _(i):
        for cv in range(D // LANES):
            zbuf[i, pl.ds(cv * LANES, LANES)] = jnp.zeros((LANES,), jnp.float32)

    @pl.loop(0, ROWS_PER_TILE // ZCHUNK)
    def _(k):
        pltpu.sync_copy(zbuf, acc.at[pl.ds(s * ROWS_PER_TILE + k * ZCHUNK, ZCHUNK)])

    @pl.when(s == NS - 1)
    def _():
        pltpu.sync_copy(zbuf.at[pl.ds(0, N_NODES - EXTRA_BASE)],
                        acc.at[pl.ds(EXTRA_BASE, N_NODES - EXTRA_BASE)])

    plsc.subcore_barrier()

    @pl.loop(0, FULL_CHUNKS)
    def _(j):
        base = ebase + j * CHUNK
        pltpu.sync_copy(src_h.at[pl.ds(base, CHUNK)], srcv)
        pltpu.sync_copy(dst_h.at[pl.ds(base, CHUNK)], dstv.at[0])
        pltpu.sync_copy(val_h.at[pl.ds(base, CHUNK)], valv)
        pltpu.async_copy(x_h.at[srcv], rows, gsem).wait()
        _scale_rows(rows, valv, CHUNK)
        pltpu.sync_copy(rows, acc.at[dstv.at[0]], add=True)

    tbase = ebase + FULL_CHUNKS * CHUNK
    pltpu.sync_copy(src_h.at[pl.ds(tbase, TAIL)], srct)
    pltpu.sync_copy(dst_h.at[pl.ds(tbase, TAIL)], dstt.at[0])
    pltpu.sync_copy(val_h.at[pl.ds(tbase, TAIL)], valt)
    pltpu.async_copy(x_h.at[srct], rowst, gsem).wait()
    _scale_rows(rowst, valt, TAIL)
    pltpu.sync_copy(rowst, acc.at[dstt.at[0]], add=True)

    plsc.subcore_barrier()
    rb = s * ROWS_PER_TILE
    pltpu.sync_copy(acc.at[pl.ds(rb, ROWS_PER_TILE)],
                    out_h.at[c, pl.ds(rb, ROWS_PER_TILE)])

    @pl.when(s == NS - 1)
    def _():
        pltpu.sync_copy(acc.at[pl.ds(EXTRA_BASE, N_NODES - EXTRA_BASE)],
                        out_h.at[c, pl.ds(EXTRA_BASE, N_NODES - EXTRA_BASE)])


_sc_scatter = pl.kernel(
    _sc_body,
    out_type=jax.ShapeDtypeStruct((NC, N_NODES, D), jnp.float32),
    mesh=plsc.VectorSubcoreMesh(
        core_axis_name="c", subcore_axis_name="s",
        num_cores=NC, num_subcores=NS),
    scratch_types=[
        pltpu.VMEM_SHARED((N_NODES, D), jnp.float32),
        pltpu.VMEM((ZCHUNK, D), jnp.float32),
        pltpu.VMEM((CHUNK,), jnp.int32),
        pltpu.VMEM((1, CHUNK), jnp.int32),
        pltpu.VMEM((CHUNK,), jnp.float32),
        pltpu.VMEM((CHUNK, D), jnp.float32),
        pltpu.VMEM((TAIL,), jnp.int32),
        pltpu.VMEM((1, TAIL), jnp.int32),
        pltpu.VMEM((TAIL,), jnp.float32),
        pltpu.VMEM((TAIL, D), jnp.float32),
        pltpu.SemaphoreType.DMA,
    ],
)


BR = 400


def _mm_body(p_ref, w_ref, b_ref, o_ref):
    z = p_ref[0] + p_ref[1]
    o_ref[...] = (
        jnp.dot(z, w_ref[...], preferred_element_type=jnp.float32) + b_ref[...]
    )


_tc_matmul = pl.pallas_call(
    _mm_body,
    grid=(N_NODES // BR,),
    in_specs=[
        pl.BlockSpec((NC, BR, D), lambda i: (0, i, 0)),
        pl.BlockSpec((D, D), lambda i: (0, 0)),
        pl.BlockSpec((1, D), lambda i: (0, 0)),
    ],
    out_specs=pl.BlockSpec((BR, D), lambda i: (i, 0)),
    out_shape=jax.ShapeDtypeStruct((N_NODES, D), jnp.float32),
)


@jax.jit
def kernel(adjacency_indices, adjacency_values, input_features, W, bias):
    dst = adjacency_indices[0]
    src = adjacency_indices[1]
    partials = _sc_scatter(src, dst, adjacency_values, input_features)
    return _tc_matmul(partials, W, bias.reshape(1, D))


# speedup vs baseline: 2.4395x; 1.2192x over previous
import functools

import jax
import jax.numpy as jnp
from jax import lax
from jax.experimental import pallas as pl
from jax.experimental.pallas import tpu as pltpu
from jax.experimental.pallas import tpu_sc as plsc

N_NODES = 10000
D = 128
N_EDGES = 320000

NC = 2
NS = 16
NW = NC * NS
LANES = 16

EDGES_PER_WORKER = N_EDGES // NW
CHUNK = 128
FULL_CHUNKS = EDGES_PER_WORKER // CHUNK
TAIL = EDGES_PER_WORKER - FULL_CHUNKS * CHUNK
ROWS_PER_TILE = 624
ZCHUNK = 208
EXTRA_BASE = ROWS_PER_TILE * NS


def _scale_rows(rows_ref, val_ref, n_edges):
    @pl.loop(0, n_edges // LANES)
    def _(g):
        vv = val_ref[pl.ds(g * LANES, LANES)]
        for l in range(LANES):
            v = vv[l]
            e = g * LANES + l
            for c in range(D // LANES):
                sl = pl.ds(c * LANES, LANES)
                rows_ref[e, sl] = rows_ref[e, sl] * v


def _sc_body(src_h, dst_h, val_h, x_h, out_h,
             acc, zbuf, srcv, dstv, valv, rows,
             srct, dstt, valt, rowst, gsem):
    c = lax.axis_index("c")
    s = lax.axis_index("s")
    wid = c * NS + s
    ebase = wid * EDGES_PER_WORKER

    @pl.loop(0, ZCHUNK)
    def _(i):
        for cv in range(D // LANES):
            zbuf[i, pl.ds(cv * LANES, LANES)] = jnp.zeros((LANES,), jnp.float32)

    @pl.loop(0, ROWS_PER_TILE // ZCHUNK)
    def _(k):
        pltpu.sync_copy(zbuf, acc.at[pl.ds(s * ROWS_PER_TILE + k * ZCHUNK, ZCHUNK)])

    @pl.when(s == NS - 1)
    def _():
        pltpu.sync_copy(zbuf.at[pl.ds(0, N_NODES - EXTRA_BASE)],
                        acc.at[pl.ds(EXTRA_BASE, N_NODES - EXTRA_BASE)])

    plsc.subcore_barrier()

    @pl.loop(0, FULL_CHUNKS)
    def _(j):
        base = ebase + j * CHUNK
        pltpu.async_copy(src_h.at[pl.ds(base, CHUNK)], srcv, gsem)
        pltpu.async_copy(dst_h.at[pl.ds(base, CHUNK)], dstv.at[0], gsem)
        pltpu.async_copy(val_h.at[pl.ds(base, CHUNK)], valv, gsem)
        pltpu.make_async_copy(src_h.at[pl.ds(base, CHUNK)], srcv, gsem).wait()
        pltpu.make_async_copy(dst_h.at[pl.ds(base, CHUNK)], dstv.at[0], gsem).wait()
        pltpu.make_async_copy(val_h.at[pl.ds(base, CHUNK)], valv, gsem).wait()
        pltpu.async_copy(x_h.at[srcv], rows, gsem).wait()
        _scale_rows(rows, valv, CHUNK)
        pltpu.sync_copy(rows, acc.at[dstv.at[0]], add=True)

    tbase = ebase + FULL_CHUNKS * CHUNK
    pltpu.sync_copy(src_h.at[pl.ds(tbase, TAIL)], srct)
    pltpu.sync_copy(dst_h.at[pl.ds(tbase, TAIL)], dstt.at[0])
    pltpu.sync_copy(val_h.at[pl.ds(tbase, TAIL)], valt)
    pltpu.async_copy(x_h.at[srct], rowst, gsem).wait()
    _scale_rows(rowst, valt, TAIL)
    pltpu.sync_copy(rowst, acc.at[dstt.at[0]], add=True)

    plsc.subcore_barrier()
    rb = s * ROWS_PER_TILE
    pltpu.sync_copy(acc.at[pl.ds(rb, ROWS_PER_TILE)],
                    out_h.at[c, pl.ds(rb, ROWS_PER_TILE)])

    @pl.when(s == NS - 1)
    def _():
        pltpu.sync_copy(acc.at[pl.ds(EXTRA_BASE, N_NODES - EXTRA_BASE)],
                        out_h.at[c, pl.ds(EXTRA_BASE, N_NODES - EXTRA_BASE)])


_sc_scatter = pl.kernel(
    _sc_body,
    out_type=jax.ShapeDtypeStruct((NC, N_NODES, D), jnp.float32),
    mesh=plsc.VectorSubcoreMesh(
        core_axis_name="c", subcore_axis_name="s",
        num_cores=NC, num_subcores=NS),
    scratch_types=[
        pltpu.VMEM_SHARED((N_NODES, D), jnp.float32),
        pltpu.VMEM((ZCHUNK, D), jnp.float32),
        pltpu.VMEM((CHUNK,), jnp.int32),
        pltpu.VMEM((1, CHUNK), jnp.int32),
        pltpu.VMEM((CHUNK,), jnp.float32),
        pltpu.VMEM((CHUNK, D), jnp.float32),
        pltpu.VMEM((TAIL,), jnp.int32),
        pltpu.VMEM((1, TAIL), jnp.int32),
        pltpu.VMEM((TAIL,), jnp.float32),
        pltpu.VMEM((TAIL, D), jnp.float32),
        pltpu.SemaphoreType.DMA,
    ],
)


BR = 400


def _mm_body(p_ref, w_ref, b_ref, o_ref):
    z = p_ref[0] + p_ref[1]
    o_ref[...] = (
        jnp.dot(z, w_ref[...], preferred_element_type=jnp.float32) + b_ref[...]
    )


_tc_matmul = pl.pallas_call(
    _mm_body,
    grid=(N_NODES // BR,),
    in_specs=[
        pl.BlockSpec((NC, BR, D), lambda i: (0, i, 0)),
        pl.BlockSpec((D, D), lambda i: (0, 0)),
        pl.BlockSpec((1, D), lambda i: (0, 0)),
    ],
    out_specs=pl.BlockSpec((BR, D), lambda i: (i, 0)),
    out_shape=jax.ShapeDtypeStruct((N_NODES, D), jnp.float32),
)


@jax.jit
def kernel(adjacency_indices, adjacency_values, input_features, W, bias):
    dst = adjacency_indices[0]
    src = adjacency_indices[1]
    partials = _sc_scatter(src, dst, adjacency_values, input_features)
    return _tc_matmul(partials, W, bias.reshape(1, D))


# speedup vs baseline: 2.7676x; 1.1345x over previous
import functools

import jax
import jax.numpy as jnp
from jax import lax
from jax.experimental import pallas as pl
from jax.experimental.pallas import tpu as pltpu
from jax.experimental.pallas import tpu_sc as plsc

N_NODES = 10000
D = 128
N_EDGES = 320000

NC = 2
NS = 16
NW = NC * NS
LANES = 16

EDGES_PER_WORKER = N_EDGES // NW
CHUNK = 128
FULL_CHUNKS = EDGES_PER_WORKER // CHUNK
TAIL = EDGES_PER_WORKER - FULL_CHUNKS * CHUNK
ROWS_PER_TILE = 624
ZCHUNK = 208
EXTRA_BASE = ROWS_PER_TILE * NS


def _scale_rows(rows_ref, val_ref, n_edges):
    @pl.loop(0, n_edges // LANES)
    def _(g):
        vv = val_ref[pl.ds(g * LANES, LANES)]
        for l in range(LANES):
            v = vv[l]
            e = g * LANES + l
            for c in range(D // LANES):
                sl = pl.ds(c * LANES, LANES)
                rows_ref[e, sl] = rows_ref[e, sl] * v


def _sc_body(src_h, dst_h, val_h, x_h, out_h,
             acc, zbuf, srcv, dstv, valv, srcv2, dstv2, valv2, rows,
             srct, dstt, valt, rowst, gsem, isem, isem2):
    c = lax.axis_index("c")
    s = lax.axis_index("s")
    wid = c * NS + s
    ebase = wid * EDGES_PER_WORKER

    @pl.loop(0, ZCHUNK)
    def _(i):
        for cv in range(D // LANES):
            zbuf[i, pl.ds(cv * LANES, LANES)] = jnp.zeros((LANES,), jnp.float32)

    @pl.loop(0, ROWS_PER_TILE // ZCHUNK)
    def _(k):
        pltpu.sync_copy(zbuf, acc.at[pl.ds(s * ROWS_PER_TILE + k * ZCHUNK, ZCHUNK)])

    @pl.when(s == NS - 1)
    def _():
        pltpu.sync_copy(zbuf.at[pl.ds(0, N_NODES - EXTRA_BASE)],
                        acc.at[pl.ds(EXTRA_BASE, N_NODES - EXTRA_BASE)])

    plsc.subcore_barrier()

    def fire_idx(j, sv, dv, vv, sem):
        base = ebase + j * CHUNK
        pltpu.async_copy(src_h.at[pl.ds(base, CHUNK)], sv, sem)
        pltpu.async_copy(dst_h.at[pl.ds(base, CHUNK)], dv.at[0], sem)
        pltpu.async_copy(val_h.at[pl.ds(base, CHUNK)], vv, sem)

    def drain_idx(j, sv, dv, vv, sem):
        base = ebase + j * CHUNK
        pltpu.make_async_copy(src_h.at[pl.ds(base, CHUNK)], sv, sem).wait()
        pltpu.make_async_copy(dst_h.at[pl.ds(base, CHUNK)], dv.at[0], sem).wait()
        pltpu.make_async_copy(val_h.at[pl.ds(base, CHUNK)], vv, sem).wait()

    def do_chunk(j, sv, dv, vv, sem):
        drain_idx(j, sv, dv, vv, sem)
        pltpu.async_copy(x_h.at[sv], rows, gsem).wait()
        _scale_rows(rows, vv, CHUNK)
        pltpu.sync_copy(rows, acc.at[dv.at[0]], add=True)

    fire_idx(0, srcv, dstv, valv, isem)

    @pl.loop(0, FULL_CHUNKS // 2)
    def _(j2):
        j = 2 * j2
        fire_idx(j + 1, srcv2, dstv2, valv2, isem2)
        do_chunk(j, srcv, dstv, valv, isem)

        @pl.when(j + 2 < FULL_CHUNKS)
        def _():
            fire_idx(j + 2, srcv, dstv, valv, isem)

        do_chunk(j + 1, srcv2, dstv2, valv2, isem2)

    tbase = ebase + FULL_CHUNKS * CHUNK
    pltpu.sync_copy(src_h.at[pl.ds(tbase, TAIL)], srct)
    pltpu.sync_copy(dst_h.at[pl.ds(tbase, TAIL)], dstt.at[0])
    pltpu.sync_copy(val_h.at[pl.ds(tbase, TAIL)], valt)
    pltpu.async_copy(x_h.at[srct], rowst, gsem).wait()
    _scale_rows(rowst, valt, TAIL)
    pltpu.sync_copy(rowst, acc.at[dstt.at[0]], add=True)

    plsc.subcore_barrier()
    rb = s * ROWS_PER_TILE
    pltpu.sync_copy(acc.at[pl.ds(rb, ROWS_PER_TILE)],
                    out_h.at[c, pl.ds(rb, ROWS_PER_TILE)])

    @pl.when(s == NS - 1)
    def _():
        pltpu.sync_copy(acc.at[pl.ds(EXTRA_BASE, N_NODES - EXTRA_BASE)],
                        out_h.at[c, pl.ds(EXTRA_BASE, N_NODES - EXTRA_BASE)])


_sc_scatter = pl.kernel(
    _sc_body,
    out_type=jax.ShapeDtypeStruct((NC, N_NODES, D), jnp.float32),
    mesh=plsc.VectorSubcoreMesh(
        core_axis_name="c", subcore_axis_name="s",
        num_cores=NC, num_subcores=NS),
    scratch_types=[
        pltpu.VMEM_SHARED((N_NODES, D), jnp.float32),
        pltpu.VMEM((ZCHUNK, D), jnp.float32),
        pltpu.VMEM((CHUNK,), jnp.int32),
        pltpu.VMEM((1, CHUNK), jnp.int32),
        pltpu.VMEM((CHUNK,), jnp.float32),
        pltpu.VMEM((CHUNK,), jnp.int32),
        pltpu.VMEM((1, CHUNK), jnp.int32),
        pltpu.VMEM((CHUNK,), jnp.float32),
        pltpu.VMEM((CHUNK, D), jnp.float32),
        pltpu.VMEM((TAIL,), jnp.int32),
        pltpu.VMEM((1, TAIL), jnp.int32),
        pltpu.VMEM((TAIL,), jnp.float32),
        pltpu.VMEM((TAIL, D), jnp.float32),
        pltpu.SemaphoreType.DMA,
        pltpu.SemaphoreType.DMA,
        pltpu.SemaphoreType.DMA,
    ],
)


BR = 400


def _mm_body(p_ref, w_ref, b_ref, o_ref):
    z = p_ref[0] + p_ref[1]
    o_ref[...] = (
        jnp.dot(z, w_ref[...], preferred_element_type=jnp.float32) + b_ref[...]
    )


_tc_matmul = pl.pallas_call(
    _mm_body,
    grid=(N_NODES // BR,),
    in_specs=[
        pl.BlockSpec((NC, BR, D), lambda i: (0, i, 0)),
        pl.BlockSpec((D, D), lambda i: (0, 0)),
        pl.BlockSpec((1, D), lambda i: (0, 0)),
    ],
    out_specs=pl.BlockSpec((BR, D), lambda i: (i, 0)),
    out_shape=jax.ShapeDtypeStruct((N_NODES, D), jnp.float32),
)


@jax.jit
def kernel(adjacency_indices, adjacency_values, input_features, W, bias):
    dst = adjacency_indices[0]
    src = adjacency_indices[1]
    partials = _sc_scatter(src, dst, adjacency_values, input_features)
    return _tc_matmul(partials, W, bias.reshape(1, D))


# speedup vs baseline: 3.3127x; 1.1969x over previous
import functools

import jax
import jax.numpy as jnp
from jax import lax
from jax.experimental import pallas as pl
from jax.experimental.pallas import tpu as pltpu
from jax.experimental.pallas import tpu_sc as plsc

N_NODES = 10000
D = 128
N_EDGES = 320000

NC = 2
NS = 16
NW = NC * NS
LANES = 16

EDGES_PER_WORKER = N_EDGES // NW
CHUNK = 128
FULL_CHUNKS = EDGES_PER_WORKER // CHUNK
TAIL = EDGES_PER_WORKER - FULL_CHUNKS * CHUNK
ROWS_PER_TILE = 624
ZCHUNK = 208
EXTRA_BASE = ROWS_PER_TILE * NS


def _scale_rows(rows_ref, val_ref, n_edges):
    @pl.loop(0, n_edges // LANES)
    def _(g):
        vv = val_ref[pl.ds(g * LANES, LANES)]
        for l in range(LANES):
            v = vv[l]
            e = g * LANES + l
            for c in range(D // LANES):
                sl = pl.ds(c * LANES, LANES)
                rows_ref[e, sl] = rows_ref[e, sl] * v


def _sc_body(src_h, dst_h, val_h, x_h, out_h,
             acc, srcv, dstv, valv, rowsA, rowsB,
             srct, dstt, valt, rowst, gsem, isems, ssems):
    c = lax.axis_index("c")
    s = lax.axis_index("s")
    wid = c * NS + s
    ebase = wid * EDGES_PER_WORKER

    rows2 = (rowsA, rowsB)

    @pl.loop(0, CHUNK)
    def _(i):
        for cv in range(D // LANES):
            rowsA[i, pl.ds(cv * LANES, LANES)] = jnp.zeros((LANES,), jnp.float32)

    rb0 = s * ROWS_PER_TILE
    for k in range(ROWS_PER_TILE // CHUNK):
        pltpu.sync_copy(rowsA, acc.at[pl.ds(rb0 + k * CHUNK, CHUNK)])
    rem = ROWS_PER_TILE % CHUNK
    pltpu.sync_copy(rowsA.at[pl.ds(0, rem)],
                    acc.at[pl.ds(rb0 + ROWS_PER_TILE - rem, rem)])

    @pl.when(s == NS - 1)
    def _():
        pltpu.sync_copy(rowsA.at[pl.ds(0, N_NODES - EXTRA_BASE)],
                        acc.at[pl.ds(EXTRA_BASE, N_NODES - EXTRA_BASE)])

    plsc.subcore_barrier()

    def fire_idx(j, j2, j4):
        base = ebase + j * CHUNK
        sem = isems.at[j2]
        pltpu.async_copy(src_h.at[pl.ds(base, CHUNK)], srcv.at[j2], sem)
        pltpu.async_copy(dst_h.at[pl.ds(base, CHUNK)], dstv.at[j4], sem)
        pltpu.async_copy(val_h.at[pl.ds(base, CHUNK)], valv.at[j2], sem)

    def drain_idx(j, j2, j4):
        base = ebase + j * CHUNK
        sem = isems.at[j2]
        pltpu.make_async_copy(src_h.at[pl.ds(base, CHUNK)], srcv.at[j2], sem).wait()
        pltpu.make_async_copy(dst_h.at[pl.ds(base, CHUNK)], dstv.at[j4], sem).wait()
        pltpu.make_async_copy(val_h.at[pl.ds(base, CHUNK)], valv.at[j2], sem).wait()

    def drain_scatter(j2, j4):
        pltpu.make_async_copy(rows2[j2], acc.at[dstv.at[j4]],
                              ssems.at[j2]).wait()

    def step(j, j2, j4, first=False, prefetch=True):
        drain_idx(j, j2, j4)
        if not first:
            drain_scatter(j2, (j4 + 2) % 4)
        rt = rows2[j2]
        pltpu.async_copy(x_h.at[srcv.at[j2]], rt, gsem).wait()

        @pl.loop(0, CHUNK // LANES)
        def _(g):
            vv = valv[j2, pl.ds(g * LANES, LANES)]
            for l in range(LANES):
                v = vv[l]
                e = g * LANES + l
                for cv in range(D // LANES):
                    sl = pl.ds(cv * LANES, LANES)
                    rt[e, sl] = rt[e, sl] * v

        if prefetch:
            fire_idx(j + 2, j2, (j4 + 2) % 4)
        pltpu.async_copy(rt, acc.at[dstv.at[j4]], ssems.at[j2], add=True)

    fire_idx(0, 0, 0)
    fire_idx(1, 1, 1)
    step(0, 0, 0, first=True)
    step(1, 1, 1, first=True)
    step(2, 0, 2)
    step(3, 1, 3)

    @pl.loop(1, FULL_CHUNKS // 4)
    def _(q):
        for t in range(4):
            step(4 * q + t, t % 2, t)

    step(FULL_CHUNKS - 2, 0, 0, prefetch=False)
    step(FULL_CHUNKS - 1, 1, 1, prefetch=False)
    drain_scatter(0, 0)
    drain_scatter(1, 1)

    tbase = ebase + FULL_CHUNKS * CHUNK
    pltpu.sync_copy(src_h.at[pl.ds(tbase, TAIL)], srct)
    pltpu.sync_copy(dst_h.at[pl.ds(tbase, TAIL)], dstt.at[0])
    pltpu.sync_copy(val_h.at[pl.ds(tbase, TAIL)], valt)
    pltpu.async_copy(x_h.at[srct], rowst, gsem).wait()
    _scale_rows(rowst, valt, TAIL)
    pltpu.sync_copy(rowst, acc.at[dstt.at[0]], add=True)

    plsc.subcore_barrier()
    rb = s * ROWS_PER_TILE
    pltpu.sync_copy(acc.at[pl.ds(rb, ROWS_PER_TILE)],
                    out_h.at[c, pl.ds(rb, ROWS_PER_TILE)])

    @pl.when(s == NS - 1)
    def _():
        pltpu.sync_copy(acc.at[pl.ds(EXTRA_BASE, N_NODES - EXTRA_BASE)],
                        out_h.at[c, pl.ds(EXTRA_BASE, N_NODES - EXTRA_BASE)])


_sc_scatter = pl.kernel(
    _sc_body,
    out_type=jax.ShapeDtypeStruct((NC, N_NODES, D), jnp.float32),
    mesh=plsc.VectorSubcoreMesh(
        core_axis_name="c", subcore_axis_name="s",
        num_cores=NC, num_subcores=NS),
    scratch_types=[
        pltpu.VMEM_SHARED((N_NODES, D), jnp.float32),
        pltpu.VMEM((2, CHUNK), jnp.int32),
        pltpu.VMEM((4, CHUNK), jnp.int32),
        pltpu.VMEM((2, CHUNK), jnp.float32),
        pltpu.VMEM((CHUNK, D), jnp.float32),
        pltpu.VMEM((CHUNK, D), jnp.float32),
        pltpu.VMEM((TAIL,), jnp.int32),
        pltpu.VMEM((1, TAIL), jnp.int32),
        pltpu.VMEM((TAIL,), jnp.float32),
        pltpu.VMEM((TAIL, D), jnp.float32),
        pltpu.SemaphoreType.DMA,
        pltpu.SemaphoreType.DMA((2,)),
        pltpu.SemaphoreType.DMA((2,)),
    ],
)


BR = 400


def _mm_body(p_ref, w_ref, b_ref, o_ref):
    z = p_ref[0] + p_ref[1]
    o_ref[...] = (
        jnp.dot(z, w_ref[...], preferred_element_type=jnp.float32) + b_ref[...]
    )


_tc_matmul = pl.pallas_call(
    _mm_body,
    grid=(N_NODES // BR,),
    in_specs=[
        pl.BlockSpec((NC, BR, D), lambda i: (0, i, 0)),
        pl.BlockSpec((D, D), lambda i: (0, 0)),
        pl.BlockSpec((1, D), lambda i: (0, 0)),
    ],
    out_specs=pl.BlockSpec((BR, D), lambda i: (i, 0)),
    out_shape=jax.ShapeDtypeStruct((N_NODES, D), jnp.float32),
)


@jax.jit
def kernel(adjacency_indices, adjacency_values, input_features, W, bias):
    dst = adjacency_indices[0]
    src = adjacency_indices[1]
    partials = _sc_scatter(src, dst, adjacency_values, input_features)
    return _tc_matmul(partials, W, bias.reshape(1, D))
